# baseline (speedup 1.0000x reference)
import functools

import jax
import jax.numpy as jnp
from jax import lax
from jax.experimental import pallas as pl
from jax.experimental.pallas import tpu as pltpu
from jax.experimental.pallas import tpu_sc as plsc

_CHUNK = 1024


def kernel(zt, pred, Qt, Qsb, Qtb, batch):
    n, c = zt.shape
    bs = Qt.shape[0]
    batch = batch.astype(jnp.int32)

    info = plsc.get_sparse_core_info()
    ncores = info.num_cores
    nw = ncores * info.num_subcores
    chunk = _CHUNK
    nchunks = n // chunk
    cpw = nchunks // nw

    bmat = batch.reshape(nchunks, chunk)
    clo = jnp.pad(bmat[:, 0], (0, 160 - nchunks))
    chi = jnp.pad(bmat[:, -1], (0, 160 - nchunks))
    starts = jnp.searchsorted(
        batch, jnp.arange(bs + 1, dtype=jnp.int32)).astype(jnp.int32)
    starts = jnp.pad(starts, (0, 512 - (bs + 1)))
    qall = jnp.concatenate(
        [Qt.reshape(bs, c * c), Qtb.reshape(bs, c * c),
         Qsb.reshape(bs, c * c)], axis=1)
    qrow = 3 * c * c + 32
    qall = jnp.pad(qall, ((0, 0), (0, qrow - 3 * c * c))).reshape(-1)
    zt_t = zt.T.reshape(c, nchunks, chunk).transpose(1, 0, 2).reshape(-1)
    pred_t = pred.T.reshape(c, nchunks, chunk).transpose(1, 0, 2).reshape(-1)

    mesh = plsc.VectorSubcoreMesh(core_axis_name="core", subcore_axis_name="sub")

    @functools.partial(
        pl.kernel,
        mesh=mesh,
        out_type=jax.ShapeDtypeStruct((n * c,), jnp.float32),
        scratch_types=[
            pltpu.VMEM((c * chunk,), jnp.float32),
            pltpu.VMEM((c * chunk,), jnp.float32),
            pltpu.VMEM((c * chunk,), jnp.float32),
            pltpu.VMEM((c * chunk,), jnp.float32),
            pltpu.VMEM((chunk,), jnp.float32),
            pltpu.VMEM((qrow,), jnp.float32),
            pltpu.VMEM((512,), jnp.int32),
            pltpu.VMEM((160,), jnp.int32),
            pltpu.VMEM((160,), jnp.int32),
        ],
    )
    def sc(zt_hbm, pred_hbm, qall_hbm, starts_hbm, clo_hbm, chi_hbm, out_hbm,
           zt_v, pred_v, out_v, e_v, isum_v, qbuf, starts_v, clo_v, chi_v):
        wid = lax.axis_index("sub") * ncores + lax.axis_index("core")
        pltpu.sync_copy(starts_hbm, starts_v)
        pltpu.sync_copy(clo_hbm, clo_v)
        pltpu.sync_copy(chi_hbm, chi_v)
        lanes = lax.iota(jnp.int32, 16)

        for k in range(cpw):
            ck = wid * cpw + k
            pltpu.sync_copy(zt_hbm.at[pl.ds(ck * c * chunk, c * chunk)], zt_v)
            pltpu.sync_copy(
                pred_hbm.at[pl.ds(ck * c * chunk, c * chunk)], pred_v)

            def softmax_g(g, carry):
                o = g * 16
                pc = [pred_v[pl.ds(cc * chunk + o, 16)] for cc in range(c)]
                m = pc[0]
                for cc in range(1, c):
                    m = jnp.maximum(m, pc[cc])
                ssum = jnp.zeros((16,), jnp.float32)
                for cc in range(c):
                    e = jnp.exp(pc[cc] - m)
                    e_v[pl.ds(cc * chunk + o, 16)] = e
                    ssum = ssum + e
                isum_v[pl.ds(o, 16)] = 1.0 / ssum
                return carry

            lax.fori_loop(0, chunk // 16, softmax_g, 0)

            def row_scalars(base):
                r0 = qbuf[pl.ds(base, 16)]
                r1 = qbuf[pl.ds(base + 16, 16)]
                return ([r0[j] for j in range(16)]
                        + [r1[j] for j in range(c - 16)])

            def seg(b, carry):
                pltpu.sync_copy(qall_hbm.at[pl.ds(b * qrow, qrow)], qbuf)
                sv = starts_v[pl.ds(b, 16)]
                cbase = ck * chunk
                lo = jnp.maximum(sv[0] - cbase, 0)
                hi = jnp.minimum(sv[1] - cbase, chunk)
                g_lo = lax.div(lo, 16)
                g_hi = lax.div(hi + 15, 16)

                def grp(g, carry2):
                    o = g * 16
                    node_l = o + lanes
                    ztv = [zt_v[pl.ds(d * chunk + o, 16)] for d in range(c)]
                    inv_ssum = isum_v[pl.ds(o, 16)]

                    s = [jnp.zeros((16,), jnp.float32) for _ in range(c)]
                    for cc in range(c):
                        qtb = row_scalars(c * c + cc * c)
                        den = qtb[0] * ztv[0]
                        for d in range(1, c):
                            den = den + qtb[d] * ztv[d]
                        den = jnp.maximum(den, 1e-6)
                        e = e_v[pl.ds(cc * chunk + o, 16)]
                        w = e * inv_ssum / den
                        qsb = row_scalars(2 * c * c + cc * c)
                        for d in range(c):
                            s[d] = s[d] + w * qsb[d]

                    un = []
                    tot = jnp.zeros((16,), jnp.float32)
                    for d in range(c):
                        qt = row_scalars(d * c)
                        left = qt[0] * ztv[0]
                        for cc in range(1, c):
                            left = left + qt[cc] * ztv[cc]
                        u = jnp.maximum(left * s[d], 1e-5)
                        un.append(u)
                        tot = tot + u
                    inv_tot = 1.0 / tot
                    mask = (node_l >= lo) & (node_l < hi)
                    for d in range(c):
                        old = out_v[pl.ds(d * chunk + o, 16)]
                        out_v[pl.ds(d * chunk + o, 16)] = jnp.where(
                            mask, un[d] * inv_tot, old)
                    return carry2

                lax.fori_loop(g_lo, g_hi, grp, 0)
                return carry

            ckv = clo_v[pl.ds(ck, 16)]
            chv = chi_v[pl.ds(ck, 16)]
            lax.fori_loop(ckv[0], chv[0] + 1, seg, 0)
            pltpu.sync_copy(out_v, out_hbm.at[pl.ds(ck * c * chunk, c * chunk)])

    out = sc(zt_t, pred_t, qall, starts, clo, chi)
    return out.reshape(nchunks, c, chunk).transpose(0, 2, 1).reshape(n, c)


# speedup vs baseline: 1.5129x; 1.5129x over previous
import functools

import jax
import jax.numpy as jnp
from jax import lax
from jax.experimental import pallas as pl
from jax.experimental.pallas import tpu as pltpu
from jax.experimental.pallas import tpu_sc as plsc

_CHUNK = 1024
_SC_CHUNKS = 32
_R = 1024
_UNROLL = 3


def _tc_block_kernel(blo_ref, bhi_ref, batch_ref, zt_ref, pred_ref,
                     qcat_ref, qsb_ref, out_ref):
    i = pl.program_id(0)
    b_lo = blo_ref[i]
    b_hi = bhi_ref[i]
    zt = zt_ref[...]
    pred = pred_ref[...]
    bidx = batch_ref[...]

    m = jnp.max(pred, axis=-1, keepdims=True)
    e = jnp.exp(pred - m)
    pred_x = e / jnp.sum(e, axis=-1, keepdims=True)

    r, c = zt.shape
    bs = qsb_ref.shape[0]

    def body(b, acc):
        bb = jnp.minimum(b, bs - 1)
        ld = jnp.dot(zt, qcat_ref[bb], preferred_element_type=jnp.float32)
        left = ld[:, :c]
        den = ld[:, c:]
        den = jnp.where(den == 0.0, 1e-6, den)
        w = pred_x / den
        s = jnp.dot(w, qsb_ref[bb], preferred_element_type=jnp.float32)
        mask = bidx == b
        return acc + jnp.where(mask, left * s, 0.0)

    acc = jnp.zeros((r, c), jnp.float32)
    for j in range(_UNROLL):
        acc = body(b_lo + j, acc)
    un = lax.fori_loop(b_lo + _UNROLL, b_hi + 1, body, acc)
    un = jnp.where(un <= 0.0, 1e-5, un)
    out_ref[...] = un / jnp.sum(un, axis=-1, keepdims=True)


def _tc_half(zt, pred, qcat, Qsb, batch):
    n, c = zt.shape
    bs = Qsb.shape[0]
    r = _R
    nb = n // r

    bmat = batch.reshape(nb, r)
    blo = bmat[:, 0]
    bhi = bmat[:, -1]
    batch2d = batch.reshape(n, 1)

    grid_spec = pltpu.PrefetchScalarGridSpec(
        num_scalar_prefetch=2,
        grid=(nb,),
        in_specs=[
            pl.BlockSpec((r, 1), lambda i, *_: (i, 0)),
            pl.BlockSpec((r, c), lambda i, *_: (i, 0)),
            pl.BlockSpec((r, c), lambda i, *_: (i, 0)),
            pl.BlockSpec((bs, c, 2 * c), lambda i, *_: (0, 0, 0)),
            pl.BlockSpec((bs, c, c), lambda i, *_: (0, 0, 0)),
        ],
        out_specs=pl.BlockSpec((r, c), lambda i, *_: (i, 0)),
    )
    return pl.pallas_call(
        _tc_block_kernel,
        grid_spec=grid_spec,
        out_shape=jax.ShapeDtypeStruct((n, c), jnp.float32),
    )(blo, bhi, batch2d, zt, pred, qcat, Qsb)


def _sc_half(zt, pred, qall, qrow, batch):
    n, c = zt.shape
    info = plsc.get_sparse_core_info()
    ncores = info.num_cores
    nw = ncores * info.num_subcores
    chunk = _CHUNK
    nchunks = n // chunk
    cpw = nchunks // nw

    bmat = batch.reshape(nchunks, chunk)
    clo = jnp.pad(bmat[:, 0], (0, 160 - nchunks))
    chi = jnp.pad(bmat[:, -1], (0, 160 - nchunks))
    bs = 256
    starts = jnp.searchsorted(
        batch, jnp.arange(bs + 1, dtype=jnp.int32)).astype(jnp.int32)
    starts = jnp.pad(starts, (0, 512 - (bs + 1)))
    zt_t = zt.T.reshape(c, nchunks, chunk).transpose(1, 0, 2).reshape(-1)
    pred_t = pred.T.reshape(c, nchunks, chunk).transpose(1, 0, 2).reshape(-1)

    mesh = plsc.VectorSubcoreMesh(
        core_axis_name="core", subcore_axis_name="sub")

    @functools.partial(
        pl.kernel,
        mesh=mesh,
        out_type=jax.ShapeDtypeStruct((n * c,), jnp.float32),
        scratch_types=[
            pltpu.VMEM((c * chunk,), jnp.float32),
            pltpu.VMEM((c * chunk,), jnp.float32),
            pltpu.VMEM((c * chunk,), jnp.float32),
            pltpu.VMEM((c * chunk,), jnp.float32),
            pltpu.VMEM((chunk,), jnp.float32),
            pltpu.VMEM((qrow,), jnp.float32),
            pltpu.VMEM((512,), jnp.int32),
            pltpu.VMEM((160,), jnp.int32),
            pltpu.VMEM((160,), jnp.int32),
        ],
    )
    def sc(zt_hbm, pred_hbm, qall_hbm, starts_hbm, clo_hbm, chi_hbm, out_hbm,
           zt_v, pred_v, out_v, e_v, isum_v, qbuf, starts_v, clo_v, chi_v):
        wid = lax.axis_index("sub") * ncores + lax.axis_index("core")
        pltpu.sync_copy(starts_hbm, starts_v)
        pltpu.sync_copy(clo_hbm, clo_v)
        pltpu.sync_copy(chi_hbm, chi_v)
        lanes = lax.iota(jnp.int32, 16)

        for k in range(cpw):
            ck = wid * cpw + k
            pltpu.sync_copy(zt_hbm.at[pl.ds(ck * c * chunk, c * chunk)], zt_v)
            pltpu.sync_copy(
                pred_hbm.at[pl.ds(ck * c * chunk, c * chunk)], pred_v)

            def softmax_g(g, carry):
                o = g * 16
                pc = [pred_v[pl.ds(cc * chunk + o, 16)] for cc in range(c)]
                m = pc[0]
                for cc in range(1, c):
                    m = jnp.maximum(m, pc[cc])
                ssum = jnp.zeros((16,), jnp.float32)
                for cc in range(c):
                    e = jnp.exp(pc[cc] - m)
                    e_v[pl.ds(cc * chunk + o, 16)] = e
                    ssum = ssum + e
                isum_v[pl.ds(o, 16)] = 1.0 / ssum
                return carry

            lax.fori_loop(0, chunk // 16, softmax_g, 0)

            def row_scalars(base):
                r0 = qbuf[pl.ds(base, 16)]
                r1 = qbuf[pl.ds(base + 16, 16)]
                return ([r0[j] for j in range(16)]
                        + [r1[j] for j in range(c - 16)])

            def seg(b, carry):
                pltpu.sync_copy(qall_hbm.at[pl.ds(b * qrow, qrow)], qbuf)
                sv = starts_v[pl.ds(b, 16)]
                cbase = ck * chunk
                lo = jnp.maximum(sv[0] - cbase, 0)
                hi = jnp.minimum(sv[1] - cbase, chunk)
                g_lo = lax.div(lo, 16)
                g_hi = lax.div(hi + 15, 16)

                def grp(g, carry2):
                    o = g * 16
                    node_l = o + lanes
                    ztv = [zt_v[pl.ds(d * chunk + o, 16)] for d in range(c)]
                    inv_ssum = isum_v[pl.ds(o, 16)]

                    s = [jnp.zeros((16,), jnp.float32) for _ in range(c)]
                    for cc in range(c):
                        qtb = row_scalars(c * c + cc * c)
                        den = qtb[0] * ztv[0]
                        for d in range(1, c):
                            den = den + qtb[d] * ztv[d]
                        den = jnp.maximum(den, 1e-6)
                        e = e_v[pl.ds(cc * chunk + o, 16)]
                        w = e * inv_ssum / den
                        qsb = row_scalars(2 * c * c + cc * c)
                        for d in range(c):
                            s[d] = s[d] + w * qsb[d]

                    un = []
                    tot = jnp.zeros((16,), jnp.float32)
                    for d in range(c):
                        qt = row_scalars(d * c)
                        left = qt[0] * ztv[0]
                        for cc in range(1, c):
                            left = left + qt[cc] * ztv[cc]
                        u = jnp.maximum(left * s[d], 1e-5)
                        un.append(u)
                        tot = tot + u
                    inv_tot = 1.0 / tot
                    mask = (node_l >= lo) & (node_l < hi)
                    for d in range(c):
                        old = out_v[pl.ds(d * chunk + o, 16)]
                        out_v[pl.ds(d * chunk + o, 16)] = jnp.where(
                            mask, un[d] * inv_tot, old)
                    return carry2

                lax.fori_loop(g_lo, g_hi, grp, 0)
                return carry

            ckv = clo_v[pl.ds(ck, 16)]
            chv = chi_v[pl.ds(ck, 16)]
            lax.fori_loop(ckv[0], chv[0] + 1, seg, 0)
            pltpu.sync_copy(
                out_v, out_hbm.at[pl.ds(ck * c * chunk, c * chunk)])

    out = sc(zt_t, pred_t, qall, starts, clo, chi)
    return out.reshape(nchunks, c, chunk).transpose(0, 2, 1).reshape(n, c)


def kernel(zt, pred, Qt, Qsb, Qtb, batch):
    n, c = zt.shape
    bs = Qt.shape[0]
    batch = batch.astype(jnp.int32)

    qcat = jnp.concatenate(
        [jnp.swapaxes(Qt, 1, 2), jnp.swapaxes(Qtb, 1, 2)], axis=2)
    qall = jnp.concatenate(
        [Qt.reshape(bs, c * c), Qtb.reshape(bs, c * c),
         Qsb.reshape(bs, c * c)], axis=1)
    qrow = 3 * c * c + 32
    qall = jnp.pad(qall, ((0, 0), (0, qrow - 3 * c * c))).reshape(-1)

    n_sc = _SC_CHUNKS * _CHUNK
    n_tc = n - n_sc

    out_sc = _sc_half(zt[n_tc:], pred[n_tc:], qall, qrow, batch[n_tc:])
    out_tc = _tc_half(zt[:n_tc], pred[:n_tc], qcat, Qsb, batch[:n_tc])
    return jnp.concatenate([out_tc, out_sc], axis=0)
